# baseline (speedup 1.0000x reference)
import jax
import jax.numpy as jnp
from jax.experimental import pallas as pl
from jax.experimental.pallas import tpu as pltpu

N = 8192
F = 1024
E = 64
T = 4
SCALE = F ** (-0.5)
TILE1 = 1024
TILE3 = 1024
PREC = jax.lax.Precision.DEFAULT
_DNT = (((1,), (1,)), ((), ()))


def _onehot_argmax(lg):
    m = jnp.max(lg, axis=1, keepdims=True)
    io = jax.lax.broadcasted_iota(jnp.int32, lg.shape, 1)
    idx = jnp.min(jnp.where(lg == m, io, E), axis=1, keepdims=True)
    return (io == idx).astype(jnp.float32)


def _phase1(f_ref, w0_ref, w1_ref, w2_ref, w3_ref, bt_ref, wb0_ref, bb0_ref,
            wb1_ref, bb1_ref, we_ref, be_ref, hm_ref, hfp_ref, csp_ref):
    f = f_ref[...]
    wb0 = wb0_ref[...]
    bb0 = bb0_ref[...]
    wb1 = wb1_ref[...]
    bb1 = bb1_ref[...]
    we = we_ref[...]
    be = be_ref[...]

    def tail(af):
        h = jax.lax.dot_general(
            jnp.maximum(af, 0.0), wb0, _DNT, precision=PREC) + bb0
        z = jax.lax.dot_general(
            jnp.maximum(h, 0.0), wb1, _DNT, precision=PREC) + bb1
        lg = jax.lax.dot_general(
            jnp.maximum(z, 0.0), we, _DNT, precision=PREC) + be
        return z, lg

    hm = jnp.zeros((TILE1, E), jnp.float32)
    for i, w_ref in enumerate((w0_ref, w1_ref, w2_ref, w3_ref)):
        af = jax.lax.dot_general(
            f, w_ref[...], _DNT, precision=PREC) + bt_ref[i]
        _, lg = tail(af)
        hm = hm + _onehot_argmax(lg)
    z0, lg0 = tail(f)
    hm = hm + _onehot_argmax(lg0)

    hm_ref[...] = hm
    mask = (hm > 0.0).astype(jnp.float32)
    hfp_ref[0] = jax.lax.dot_general(
        mask, z0, (((0,), (0,)), ((), ())), precision=PREC)
    csp_ref[0] = jnp.sum(jnp.exp(hm), axis=0, keepdims=True)


def _phase2(f_ref, hfp_ref, csp_ref, w0_ref, w1_ref, w2_ref, w3_ref, bt_ref,
            hm_ref, hf_ref, dots_ref, hs_ref, g_ref, c_ref, cs_ref):
    if True:
        hf = jnp.sum(hfp_ref[...], axis=0)
        hf_ref[...] = hf
        cs_ref[...] = jnp.sum(csp_ref[...], axis=0)
        g_ref[0:E] = hf
        c_ref[:, 0:E] = jnp.zeros((1, E), jnp.float32)
        for i, w_ref in enumerate((w0_ref, w1_ref, w2_ref, w3_ref)):
            g_ref[(i + 1) * E:(i + 2) * E] = jnp.dot(
                hf, w_ref[...], precision=PREC)
            c_ref[:, (i + 1) * E:(i + 2) * E] = jax.lax.dot_general(
                bt_ref[i], hf, _DNT, precision=PREC)

    hs_ref[...] = jnp.exp(hm_ref[...]) / cs_ref[...]
    f = f_ref[...]
    c = c_ref[...]
    d01 = (jax.lax.dot_general(f, g_ref[0:2 * E], _DNT, precision=PREC)
           + c[:, 0:2 * E]) * SCALE
    dots_ref[0] = d01[:, :E]
    dots_ref[1] = d01[:, E:]
    d23 = (jax.lax.dot_general(f, g_ref[2 * E:4 * E], _DNT, precision=PREC)
           + c[:, 2 * E:4 * E]) * SCALE
    dots_ref[2] = d23[:, :E]
    dots_ref[3] = d23[:, E:]
    d4 = (jax.lax.dot_general(f, g_ref[4 * E:5 * E], _DNT, precision=PREC)
          + c[:, 4 * E:5 * E]) * SCALE
    dots_ref[4] = d4


def kernel(features, W0, b0, W1, b1, W2, b2, W3, b3, Wb0, bb0, Wb1, bb1, We, be):
    bst = jnp.stack([b0, b1, b2, b3])[:, None, :]
    bb0r = bb0[None, :]
    bb1r = bb1[None, :]
    ber = be[None, :]

    wspec = pl.BlockSpec((F, F), lambda i: (0, 0))
    rt1 = N // TILE1
    hm, hfp, csp = pl.pallas_call(
        _phase1,
        grid=(rt1,),
        in_specs=[
            pl.BlockSpec((TILE1, F), lambda i: (i, 0)),
            wspec, wspec, wspec, wspec,
            pl.BlockSpec((T, 1, F), lambda i: (0, 0, 0)),
            wspec,
            pl.BlockSpec((1, F), lambda i: (0, 0)),
            wspec,
            pl.BlockSpec((1, F), lambda i: (0, 0)),
            pl.BlockSpec((E, F), lambda i: (0, 0)),
            pl.BlockSpec((1, E), lambda i: (0, 0)),
        ],
        out_specs=[
            pl.BlockSpec((TILE1, E), lambda i: (i, 0)),
            pl.BlockSpec((1, E, F), lambda i: (i, 0, 0)),
            pl.BlockSpec((1, 1, E), lambda i: (i, 0, 0)),
        ],
        out_shape=[
            jax.ShapeDtypeStruct((N, E), jnp.float32),
            jax.ShapeDtypeStruct((rt1, E, F), jnp.float32),
            jax.ShapeDtypeStruct((rt1, 1, E), jnp.float32),
        ],
        compiler_params=pltpu.CompilerParams(
            dimension_semantics=("arbitrary",)),
    )(features, W0, W1, W2, W3, bst, Wb0, bb0r, Wb1, bb1r, We, ber)

    rt3 = N // TILE3
    hf, dots5, hs = pl.pallas_call(
        _phase2,
        grid=(rt3,),
        in_specs=[
            pl.BlockSpec((TILE3, F), lambda i: (i, 0)),
            pl.BlockSpec((rt1, E, F), lambda i: (0, 0, 0)),
            pl.BlockSpec((rt1, 1, E), lambda i: (0, 0, 0)),
            wspec, wspec, wspec, wspec,
            pl.BlockSpec((T, 1, F), lambda i: (0, 0, 0)),
            pl.BlockSpec((TILE3, E), lambda i: (i, 0)),
        ],
        out_specs=[
            pl.BlockSpec((E, F), lambda i: (0, 0)),
            pl.BlockSpec((T + 1, TILE3, E), lambda i: (0, i, 0)),
            pl.BlockSpec((TILE3, E), lambda i: (i, 0)),
        ],
        out_shape=[
            jax.ShapeDtypeStruct((E, F), jnp.float32),
            jax.ShapeDtypeStruct((T + 1, N, E), jnp.float32),
            jax.ShapeDtypeStruct((N, E), jnp.float32),
        ],
        scratch_shapes=[
            pltpu.VMEM(((T + 1) * E, F), jnp.float32),
            pltpu.VMEM((1, (T + 1) * E), jnp.float32),
            pltpu.VMEM((1, E), jnp.float32),
        ],
        compiler_params=pltpu.CompilerParams(
            dimension_semantics=("arbitrary",)),
    )(features, hfp, csp, W0, W1, W2, W3, bst, hm)

    dots = dots5.reshape((T + 1) * N, E)
    return (hs, hf, dots)


# speedup vs baseline: 1.0218x; 1.0218x over previous
import jax
import jax.numpy as jnp
from jax.experimental import pallas as pl
from jax.experimental.pallas import tpu as pltpu

N = 8192
F = 1024
E = 64
T = 4
SCALE = F ** (-0.5)
TILE1 = 1024
TILE3 = 1024
PREC = jax.lax.Precision.DEFAULT
_DNT = (((1,), (1,)), ((), ()))


def _onehot_argmax(lg):
    m = jnp.max(lg, axis=1, keepdims=True)
    io = jax.lax.broadcasted_iota(jnp.int32, lg.shape, 1)
    idx = jnp.min(jnp.where(lg == m, io, E), axis=1, keepdims=True)
    return (io == idx).astype(jnp.float32)


def _phase1(f_ref, w0_ref, w1_ref, w2_ref, w3_ref, bt_ref, wb0_ref, bb0_ref,
            wb1_ref, bb1_ref, we_ref, be_ref, hm_ref, hfp_ref, csp_ref):
    f = f_ref[...]
    wb0 = wb0_ref[...]
    bb0 = bb0_ref[...]
    wb1 = wb1_ref[...]
    bb1 = bb1_ref[...]
    we = we_ref[...]
    be = be_ref[...]

    def tail(af):
        h = jax.lax.dot_general(
            jnp.maximum(af, 0.0), wb0, _DNT, precision=PREC) + bb0
        z = jax.lax.dot_general(
            jnp.maximum(h, 0.0), wb1, _DNT, precision=PREC) + bb1
        lg = jax.lax.dot_general(
            jnp.maximum(z, 0.0), we, _DNT, precision=PREC) + be
        return z, lg

    hm = jnp.zeros((TILE1, E), jnp.float32)
    for i, w_ref in enumerate((w0_ref, w1_ref, w2_ref, w3_ref)):
        af = jax.lax.dot_general(
            f, w_ref[...], _DNT, precision=PREC) + bt_ref[i]
        _, lg = tail(af)
        hm = hm + _onehot_argmax(lg)
    z0, lg0 = tail(f)
    hm = hm + _onehot_argmax(lg0)

    hm_ref[...] = hm
    mask = (hm > 0.0).astype(jnp.float32)
    hfp_ref[0] = jax.lax.dot_general(
        mask, z0, (((0,), (0,)), ((), ())), precision=PREC)
    csp_ref[0] = jnp.sum(jnp.exp(hm), axis=0, keepdims=True)


def _phase2(f_ref, hfp_ref, csp_ref, w0_ref, w1_ref, w2_ref, w3_ref, bt_ref,
            hm_ref, hf_ref, dots_ref, hs_ref, g_ref, c_ref, cs_ref):
    @pl.when(pl.program_id(0) == 0)
    def _prep():
        hf = jnp.sum(hfp_ref[...], axis=0)
        cs_ref[...] = jnp.sum(csp_ref[...], axis=0)
        g_ref[0:E] = hf
        c_ref[:, 0:E] = jnp.zeros((1, E), jnp.float32)
        for i, w_ref in enumerate((w0_ref, w1_ref, w2_ref, w3_ref)):
            g_ref[(i + 1) * E:(i + 2) * E] = jnp.dot(
                hf, w_ref[...], precision=PREC)
            c_ref[:, (i + 1) * E:(i + 2) * E] = jax.lax.dot_general(
                bt_ref[i], hf, _DNT, precision=PREC)

    hf_ref[...] = g_ref[0:E]
    hs_ref[...] = jnp.exp(hm_ref[...]) / cs_ref[...]
    f = f_ref[...]
    c = c_ref[...]
    d01 = (jax.lax.dot_general(f, g_ref[0:2 * E], _DNT, precision=PREC)
           + c[:, 0:2 * E]) * SCALE
    dots_ref[0] = d01[:, :E]
    dots_ref[1] = d01[:, E:]
    d23 = (jax.lax.dot_general(f, g_ref[2 * E:4 * E], _DNT, precision=PREC)
           + c[:, 2 * E:4 * E]) * SCALE
    dots_ref[2] = d23[:, :E]
    dots_ref[3] = d23[:, E:]
    d4 = (jax.lax.dot_general(f, g_ref[4 * E:5 * E], _DNT, precision=PREC)
          + c[:, 4 * E:5 * E]) * SCALE
    dots_ref[4] = d4


def kernel(features, W0, b0, W1, b1, W2, b2, W3, b3, Wb0, bb0, Wb1, bb1, We, be):
    bst = jnp.stack([b0, b1, b2, b3])[:, None, :]
    bb0r = bb0[None, :]
    bb1r = bb1[None, :]
    ber = be[None, :]

    wspec = pl.BlockSpec((F, F), lambda i: (0, 0))
    rt1 = N // TILE1
    hm, hfp, csp = pl.pallas_call(
        _phase1,
        grid=(rt1,),
        in_specs=[
            pl.BlockSpec((TILE1, F), lambda i: (i, 0)),
            wspec, wspec, wspec, wspec,
            pl.BlockSpec((T, 1, F), lambda i: (0, 0, 0)),
            wspec,
            pl.BlockSpec((1, F), lambda i: (0, 0)),
            wspec,
            pl.BlockSpec((1, F), lambda i: (0, 0)),
            pl.BlockSpec((E, F), lambda i: (0, 0)),
            pl.BlockSpec((1, E), lambda i: (0, 0)),
        ],
        out_specs=[
            pl.BlockSpec((TILE1, E), lambda i: (i, 0)),
            pl.BlockSpec((1, E, F), lambda i: (i, 0, 0)),
            pl.BlockSpec((1, 1, E), lambda i: (i, 0, 0)),
        ],
        out_shape=[
            jax.ShapeDtypeStruct((N, E), jnp.float32),
            jax.ShapeDtypeStruct((rt1, E, F), jnp.float32),
            jax.ShapeDtypeStruct((rt1, 1, E), jnp.float32),
        ],
        compiler_params=pltpu.CompilerParams(
            dimension_semantics=("arbitrary",)),
    )(features, W0, W1, W2, W3, bst, Wb0, bb0r, Wb1, bb1r, We, ber)

    rt3 = N // TILE3
    hf, dots5, hs = pl.pallas_call(
        _phase2,
        grid=(rt3,),
        in_specs=[
            pl.BlockSpec((TILE3, F), lambda i: (i, 0)),
            pl.BlockSpec((rt1, E, F), lambda i: (0, 0, 0)),
            pl.BlockSpec((rt1, 1, E), lambda i: (0, 0, 0)),
            wspec, wspec, wspec, wspec,
            pl.BlockSpec((T, 1, F), lambda i: (0, 0, 0)),
            pl.BlockSpec((TILE3, E), lambda i: (i, 0)),
        ],
        out_specs=[
            pl.BlockSpec((E, F), lambda i: (0, 0)),
            pl.BlockSpec((T + 1, TILE3, E), lambda i: (0, i, 0)),
            pl.BlockSpec((TILE3, E), lambda i: (i, 0)),
        ],
        out_shape=[
            jax.ShapeDtypeStruct((E, F), jnp.float32),
            jax.ShapeDtypeStruct((T + 1, N, E), jnp.float32),
            jax.ShapeDtypeStruct((N, E), jnp.float32),
        ],
        scratch_shapes=[
            pltpu.VMEM(((T + 1) * E, F), jnp.float32),
            pltpu.VMEM((1, (T + 1) * E), jnp.float32),
            pltpu.VMEM((1, E), jnp.float32),
        ],
        compiler_params=pltpu.CompilerParams(
            dimension_semantics=("arbitrary",)),
    )(features, hfp, csp, W0, W1, W2, W3, bst, hm)

    dots = dots5.reshape((T + 1) * N, E)
    return (hs, hf, dots)
